# baseline (speedup 1.0000x reference)
import functools

import jax
import jax.numpy as jnp
from jax import lax
from jax.experimental import pallas as pl
from jax.experimental.pallas import tpu as pltpu
from jax.experimental.pallas import tpu_sc as plsc

LANES = 16


@functools.partial(jax.jit, static_argnames=("batch", "m", "ksub", "dsub"))
def _pq_gather(codes_x, table_y, *, batch, m, ksub, dsub):
    info = plsc.get_sparse_core_info()
    nc, ns = info.num_cores, info.num_subcores
    nw = nc * ns
    total = batch * m
    per_w = total // nw
    rows_per_sb = 32
    sbi = rows_per_sb * m
    nsb = per_w // sbi
    groups = m // LANES
    rtiles = m // 8
    kc = ksub // 128
    m_sc = m // ns

    mesh = plsc.VectorSubcoreMesh(core_axis_name="c", subcore_axis_name="s")

    @functools.partial(
        pl.kernel,
        mesh=mesh,
        compiler_params=pltpu.CompilerParams(
            use_tc_tiling_on_sc=False, needs_layout_passes=False
        ),
        out_type=jax.ShapeDtypeStruct((total, dsub), jnp.float32),
        scratch_types=[
            pltpu.VMEM((sbi,), jnp.int32),
            pltpu.VMEM((sbi,), jnp.int32),
            pltpu.VMEM((rtiles, 8, 128), jnp.int32),
            pltpu.VMEM((rtiles, 8, 128), jnp.int32),
            pltpu.VMEM((sbi, dsub), jnp.float32),
            pltpu.VMEM((sbi, dsub), jnp.float32),
            pltpu.VMEM((sbi, dsub), jnp.float32),
            pltpu.VMEM((m_sc, kc, dsub, 128), jnp.float32),
            pltpu.VMEM_SHARED((m * ksub, dsub), jnp.float32),
            pltpu.SemaphoreType.DMA,
            pltpu.SemaphoreType.DMA,
            pltpu.SemaphoreType.DMA,
            pltpu.SemaphoreType.DMA,
            pltpu.SemaphoreType.DMA,
            pltpu.SemaphoreType.DMA,
            pltpu.SemaphoreType.DMA,
        ],
    )
    def k(codes_hbm, table_hbm, out_hbm, idx0, idx1, cbufa, cbufb, dat0,
          dat1, dat2, ybuf, table_sh, gsem0, gsem1, gsem2,
          wsem0, wsem1, wsem2, csem):
        cid = lax.axis_index("c")
        sid = lax.axis_index("s")
        wid = sid * nc + cid
        base = wid * per_w

        iota = lax.iota(jnp.int32, LANES)
        iota_div8 = lax.shift_right_logical(iota, 3)
        iota_mod8 = lax.bitwise_and(iota, 7)

        cbufs = (cbufa, cbufb)
        base_tile = wid * (per_w // m // 128)

        def fetch_start(tc, buf):
            for rt in range(rtiles):
                pltpu.async_copy(
                    codes_hbm.at[rt, base_tile + tc], buf.at[rt], csem
                )

        def fetch_wait(tc, buf):
            for rt in range(rtiles):
                pltpu.make_async_copy(
                    codes_hbm.at[rt, base_tile + tc], buf.at[rt], csem
                ).wait()

        fetch_start(0, cbufs[0])

        pltpu.sync_copy(table_hbm.at[pl.ds(sid * m_sc, m_sc)], ybuf)

        def stage_j(j, carry):
            for cc in range(kc):
                for d in range(dsub):
                    for w in range(128 // LANES):
                        vals = ybuf[j, cc, d, pl.ds(w * LANES, LANES)]
                        i0 = iota + (j * ksub + cc * 128 + w * LANES)
                        i1 = jnp.full((LANES,), d, jnp.int32)
                        plsc.store_scatter(dat0, [i0, i1], vals)
            return carry

        lax.fori_loop(0, m_sc, stage_j, 0)
        pltpu.sync_copy(
            dat0.at[pl.ds(0, m_sc * ksub), :],
            table_sh.at[pl.ds(sid * (m_sc * ksub), m_sc * ksub), :],
        )
        plsc.subcore_barrier()

        idx = (idx0, idx1)
        dat = (dat0, dat1, dat2)
        gsem = (gsem0, gsem1, gsem2)
        wsem = (wsem0, wsem1, wsem2)

        def prep(sb, b, buf):
            c_off = (sb % 4) * rows_per_sb

            def row_body(r, carry):
                rhi = lax.shift_right_logical(r, 3)
                rlo = lax.bitwise_and(r, 7)
                csplat = jnp.full((LANES,), c_off + r, jnp.int32)
                for g in range(groups):
                    i0 = iota_div8 + (2 * g)
                    codes_vec = plsc.load_gather(
                        buf, [i0, iota_mod8, csplat]
                    )
                    offv = iota * ksub + g * (LANES * ksub)
                    t = (rhi * groups + g) * 8 + rlo
                    idx[b][pl.ds(t * LANES, LANES)] = codes_vec + offv
                return carry

            lax.fori_loop(0, rows_per_sb, row_body, 0)

        def gather_start(sb, bi, bd):
            pltpu.async_copy(table_sh.at[idx[bi]], dat[bd], gsem[bd])

        def gather_wait(bi, bd):
            pltpu.make_async_copy(
                table_sh.at[idx[bi]], dat[bd], gsem[bd]
            ).wait()

        def write_start(sb, b):
            off = base + sb * sbi
            pltpu.async_copy(dat[b], out_hbm.at[pl.ds(off, sbi)], wsem[b])

        def write_wait(sb, b):
            off = base + sb * sbi
            pltpu.make_async_copy(dat[b], out_hbm.at[pl.ds(off, sbi)], wsem[b]).wait()

        ntiles = nsb // 4
        fetch_wait(0, cbufs[0])
        prep(0, 0, cbufs[0])
        gather_start(0, 0, 0)
        for tc in range(ntiles):
            if tc + 1 < ntiles:
                fetch_start(tc + 1, cbufs[(tc + 1) % 2])
            for ss in range(4):
                cur = tc * 4 + ss
                nxt = cur + 1
                if nxt < nsb:
                    nbuf = cbufs[(nxt // 4) % 2]
                    if nxt % 4 == 0:
                        fetch_wait(nxt // 4, nbuf)
                    prep(nxt, nxt % 2, nbuf)
                    if nxt >= 3:
                        write_wait(nxt - 3, (nxt - 3) % 3)
                    gather_start(nxt, nxt % 2, nxt % 3)
                gather_wait(cur % 2, cur % 3)
                write_start(cur, cur % 3)
        for tail in (nsb - 3, nsb - 2, nsb - 1):
            write_wait(tail, tail % 3)

    return k(codes_x, table_y)


def kernel(doc_codes, tables):
    batch, m = doc_codes.shape
    _, ksub, dsub = tables.shape
    codes = doc_codes.astype(jnp.int32)
    codes_x = (
        codes.T.reshape(m // 8, 8, batch // 128, 128).transpose(0, 2, 1, 3)
    )
    table_y = tables.reshape(m, ksub // 128, 128, dsub).transpose(0, 1, 3, 2)
    out = _pq_gather(codes_x, table_y, batch=batch, m=m, ksub=ksub, dsub=dsub)
    ngrp = m * dsub // 128
    out = out.reshape(batch // 8, ngrp, 8, 128).transpose(0, 2, 1, 3)
    return out.reshape(batch, m * dsub)
